# baseline (speedup 1.0000x reference)
import functools

import jax
import jax.numpy as jnp
from jax import lax
from jax.experimental import pallas as pl
from jax.experimental.pallas import tpu as pltpu
from jax.experimental.pallas import tpu_sc as plsc

N_TOK = 32768
DIM = 192
M = 4096
HALF = N_TOK // 2

NC = 2
NS = 16
NW = NC * NS
SPW = M // NW
CHUNK = 128
NJ = DIM // 16
NB = M + 1

_mesh = plsc.VectorSubcoreMesh(core_axis_name="c", subcore_axis_name="s")


def _sread(ref, i):
    return ref[pl.ds(i, 16)][0]


def _make_partial(t0):

    @functools.partial(
        pl.kernel,
        mesh=_mesh,
        out_type=jax.ShapeDtypeStruct((M, DIM), jnp.float32),
        scratch_types=[
            pltpu.VMEM((NB + 31,), jnp.int32),
            pltpu.VMEM((2, CHUNK, DIM), jnp.float32),
            pltpu.VMEM((SPW, DIM), jnp.float32),
            pltpu.SemaphoreType.DMA,
            pltpu.SemaphoreType.DMA,
        ],
    )
    def _partial_sum(enc_hbm, bnd_hbm, out_hbm, bnd_v, buf_v, acc_v, sem0, sem1):
        wid = lax.axis_index("s") * NC + lax.axis_index("c")
        base = wid * SPW
        pltpu.sync_copy(bnd_hbm, bnd_v.at[pl.ds(0, NB)])

        zero = jnp.zeros((16,), jnp.float32)

        def clip(x):
            return jnp.clip(x - t0, 0, HALF)

        s0 = clip(_sread(bnd_v, base))
        cs0 = jnp.minimum((s0 // 8) * 8, HALF - CHUNK)
        c1 = jnp.minimum(cs0 + CHUNK, HALF - CHUNK)
        first = pltpu.async_copy(
            enc_hbm.at[pl.ds(pl.multiple_of(cs0, 8), CHUNK)], buf_v.at[0], sem0
        )
        pltpu.async_copy(
            enc_hbm.at[pl.ds(pl.multiple_of(c1, 8), CHUNK)], buf_v.at[1], sem1
        )
        first.wait()

        def process_span(g_lo, g_hi, cs, par, accs):
            def row_body(g, a):
                local = g - cs
                return tuple(
                    a[j] + buf_v[par, local, pl.ds(j * 16, 16)]
                    for j in range(NJ)
                )

            return lax.fori_loop(g_lo, g_hi, row_body, accs)

        def advance(cs, par):
            new_cs = jnp.minimum(cs + CHUNK, HALF - CHUNK)
            new_par = 1 - par
            nxt = pl.multiple_of(jnp.minimum(new_cs + CHUNK, HALF - CHUNK), 8)

            @pl.when(new_par == 0)
            def _():
                pltpu.make_async_copy(
                    enc_hbm.at[pl.ds(0, CHUNK)], buf_v.at[0], sem0
                ).wait()
                pltpu.async_copy(
                    enc_hbm.at[pl.ds(nxt, CHUNK)], buf_v.at[1], sem1
                )

            @pl.when(new_par == 1)
            def _():
                pltpu.make_async_copy(
                    enc_hbm.at[pl.ds(0, CHUNK)], buf_v.at[1], sem1
                ).wait()
                pltpu.async_copy(
                    enc_hbm.at[pl.ds(nxt, CHUNK)], buf_v.at[0], sem0
                )

            return new_cs, new_par

        def seg_body(i, carry):
            cs, par, s = carry
            e = clip(_sread(bnd_v, base + i + 1))
            n_loads = jnp.maximum(0, (e - cs - 1) // CHUNK)
            hi = jnp.minimum(e, cs + CHUNK)
            accs = process_span(jnp.maximum(s, cs), hi, cs, par, (zero,) * NJ)

            def load_body(t, c2):
                cs2, par2, g2 = c2[0], c2[1], c2[2]
                cs2, par2 = advance(cs2, par2)
                hi2 = jnp.minimum(e, cs2 + CHUNK)
                accs2 = process_span(g2, hi2, cs2, par2, c2[3:])
                return (cs2, par2, hi2) + accs2

            res = lax.fori_loop(0, n_loads, load_body, (cs, par, hi) + accs)
            cs, par, accs = res[0], res[1], res[3:]

            for j in range(NJ):
                acc_v[i, pl.ds(j * 16, 16)] = accs[j]
            return (cs, par, e)

        end = lax.fori_loop(0, SPW, seg_body, (cs0, jnp.int32(0), s0))
        end_par = end[1]

        @pl.when(end_par == 0)
        def _():
            pltpu.make_async_copy(
                enc_hbm.at[pl.ds(0, CHUNK)], buf_v.at[1], sem1
            ).wait()

        @pl.when(end_par == 1)
        def _():
            pltpu.make_async_copy(
                enc_hbm.at[pl.ds(0, CHUNK)], buf_v.at[0], sem0
            ).wait()

        pltpu.sync_copy(acc_v, out_hbm.at[pl.ds(base, SPW)])

    return _partial_sum


_partial_lo = _make_partial(0)
_partial_hi = _make_partial(HALF)


def kernel(enc_seq, boundaries):
    bnd = boundaries.astype(jnp.int32)
    p_lo = _partial_lo(enc_seq[:HALF], bnd)
    p_hi = _partial_hi(enc_seq[HALF:], bnd)
    cnt = (bnd[1:] - bnd[:-1]).astype(jnp.float32)
    return (p_lo + p_hi) / jnp.maximum(cnt, 1.0)[:, None]


# speedup vs baseline: 1.5217x; 1.5217x over previous
import functools

import jax
import jax.numpy as jnp
from jax import lax
from jax.experimental import pallas as pl
from jax.experimental.pallas import tpu as pltpu
from jax.experimental.pallas import tpu_sc as plsc

N_TOK = 32768
DIM = 192
M = 4096

NC = 2
NS = 16
NW = NC * NS
SPW = M // NW
CHUNK = 128
NJ = DIM // 16
NB = M + 1

_mesh = plsc.VectorSubcoreMesh(core_axis_name="c", subcore_axis_name="s")


def _sread(ref, i):
    return ref[pl.ds(i, 16)][0]


@functools.partial(
    pl.kernel,
    mesh=_mesh,
    out_type=jax.ShapeDtypeStruct((M, DIM), jnp.float32),
    scratch_types=[
        pltpu.VMEM((NB + 31,), jnp.int32),
        pltpu.VMEM((2, CHUNK, DIM), jnp.float32),
        pltpu.VMEM((SPW, DIM), jnp.float32),
        pltpu.VMEM((SPW + 16,), jnp.float32),
        pltpu.SemaphoreType.DMA,
        pltpu.SemaphoreType.DMA,
    ],
)
def _seg_mean(enc_hbm, bnd_hbm, out_hbm, bnd_v, buf_v, acc_v, inv_v, sem0, sem1):
    wid = lax.axis_index("s") * NC + lax.axis_index("c")
    base = wid * SPW
    pltpu.sync_copy(bnd_hbm, bnd_v.at[pl.ds(0, NB)])

    zero = jnp.zeros((16,), jnp.float32)
    one = jnp.full((16,), 1.0, jnp.float32)

    def inv_body(k, carry):
        a = bnd_v[pl.ds(base + k * 16, 16)]
        b = bnd_v[pl.ds(base + k * 16 + 1, 16)]
        cnt = (b - a).astype(jnp.float32)
        inv_v[pl.ds(k * 16, 16)] = one / jnp.maximum(cnt, one)
        return carry

    lax.fori_loop(0, SPW // 16, inv_body, 0)

    s0 = _sread(bnd_v, base)
    cs0 = jnp.minimum((s0 // 8) * 8, N_TOK - CHUNK)
    c1 = jnp.minimum(cs0 + CHUNK, N_TOK - CHUNK)
    first = pltpu.async_copy(
        enc_hbm.at[pl.ds(pl.multiple_of(cs0, 8), CHUNK)], buf_v.at[0], sem0
    )
    pltpu.async_copy(
        enc_hbm.at[pl.ds(pl.multiple_of(c1, 8), CHUNK)], buf_v.at[1], sem1
    )
    first.wait()

    def process_span(g_lo, g_hi, cs, par, accs):
        def row_body(g, a):
            local = g - cs
            return tuple(
                a[j] + buf_v[par, local, pl.ds(j * 16, 16)] for j in range(NJ)
            )

        return lax.fori_loop(g_lo, g_hi, row_body, accs)

    def advance(cs, par):
        new_cs = jnp.minimum(cs + CHUNK, N_TOK - CHUNK)
        new_par = 1 - par
        nxt = pl.multiple_of(jnp.minimum(new_cs + CHUNK, N_TOK - CHUNK), 8)

        @pl.when(new_par == 0)
        def _():
            pltpu.make_async_copy(
                enc_hbm.at[pl.ds(0, CHUNK)], buf_v.at[0], sem0
            ).wait()
            pltpu.async_copy(enc_hbm.at[pl.ds(nxt, CHUNK)], buf_v.at[1], sem1)

        @pl.when(new_par == 1)
        def _():
            pltpu.make_async_copy(
                enc_hbm.at[pl.ds(0, CHUNK)], buf_v.at[1], sem1
            ).wait()
            pltpu.async_copy(enc_hbm.at[pl.ds(nxt, CHUNK)], buf_v.at[0], sem0)

        return new_cs, new_par

    def seg_body(i, carry):
        cs, par, s = carry
        e = _sread(bnd_v, base + i + 1)
        n_loads = jnp.maximum(0, (e - cs - 1) // CHUNK)
        hi = jnp.minimum(e, cs + CHUNK)
        accs = process_span(jnp.maximum(s, cs), hi, cs, par, (zero,) * NJ)

        def load_body(t, c2):
            cs2, par2, g2 = c2[0], c2[1], c2[2]
            cs2, par2 = advance(cs2, par2)
            hi2 = jnp.minimum(e, cs2 + CHUNK)
            accs2 = process_span(g2, hi2, cs2, par2, c2[3:])
            return (cs2, par2, hi2) + accs2

        res = lax.fori_loop(0, n_loads, load_body, (cs, par, hi) + accs)
        cs, par, accs = res[0], res[1], res[3:]

        invv = jnp.full((16,), inv_v[pl.ds(i, 16)][0], jnp.float32)
        for j in range(NJ):
            acc_v[i, pl.ds(j * 16, 16)] = accs[j] * invv
        return (cs, par, e)

    end_cs, end_par, _ = lax.fori_loop(0, SPW, seg_body, (cs0, jnp.int32(0), s0))

    @pl.when(end_par == 0)
    def _():
        pltpu.make_async_copy(
            enc_hbm.at[pl.ds(0, CHUNK)], buf_v.at[1], sem1
        ).wait()

    @pl.when(end_par == 1)
    def _():
        pltpu.make_async_copy(
            enc_hbm.at[pl.ds(0, CHUNK)], buf_v.at[0], sem0
        ).wait()

    pltpu.sync_copy(acc_v, out_hbm.at[pl.ds(base, SPW)])


def kernel(enc_seq, boundaries):
    return _seg_mean(enc_seq, boundaries.astype(jnp.int32))
